# baseline (speedup 1.0000x reference)
import functools
import math

import jax
import jax.numpy as jnp
from jax import lax
from jax.experimental import pallas as pl
from jax.experimental.pallas import tpu as pltpu
from jax.experimental.pallas import tpu_sc as plsc

N = 2048
D = 768
E = 8
DFF = 1024
H = 12
DH = 64
TOPK = 2

BM = 512
GTILE = 256
SLOTS = N * TOPK
PAD = SLOTS + E * GTILE
NTILES = PAD // GTILE
NEG = -1e30


def _ln(x, g, b, eps=1e-5):
    m = jnp.mean(x, axis=-1, keepdims=True)
    v = jnp.mean((x - m) ** 2, axis=-1, keepdims=True)
    return (x - m) * lax.rsqrt(v + eps) * g + b


def _gconv_qkv_body(adj_ref, x_ref, xt_ref, gcw_ref, gcb_ref, root_ref,
                    inw_ref, inb_ref, x1_ref, qb_ref, kb_ref, vb_ref):
    bf = jnp.bfloat16
    a = jnp.dot(adj_ref[...].astype(bf), x_ref[...].astype(bf),
                preferred_element_type=jnp.float32)
    g = lax.dot_general(a.astype(bf), gcw_ref[...].astype(bf),
                        (((1,), (1,)), ((), ())),
                        preferred_element_type=jnp.float32)
    g = g + gcb_ref[...] + root_ref[...]
    x1 = xt_ref[...] + g
    x1_ref[...] = x1
    qkv = lax.dot_general(x1.astype(bf), inw_ref[...].astype(bf),
                          (((1,), (1,)), ((), ())),
                          preferred_element_type=jnp.float32) + inb_ref[...]
    qb_ref[...] = (qkv[:, :D] * (1.0 / math.sqrt(DH))).astype(bf)
    kb_ref[...] = qkv[:, D:2 * D].astype(bf)
    vb_ref[...] = qkv[:, 2 * D:].astype(bf)


def _gconv_qkv(adj, x, gc_W, gc_b, root, in_w, in_b, interpret=False):
    return pl.pallas_call(
        _gconv_qkv_body,
        grid=(N // BM,),
        in_specs=[
            pl.BlockSpec((BM, N), lambda i: (i, 0)),
            pl.BlockSpec((N, D), lambda i: (0, 0)),
            pl.BlockSpec((BM, D), lambda i: (i, 0)),
            pl.BlockSpec((D, D), lambda i: (0, 0)),
            pl.BlockSpec((1, D), lambda i: (0, 0)),
            pl.BlockSpec((1, D), lambda i: (0, 0)),
            pl.BlockSpec((3 * D, D), lambda i: (0, 0)),
            pl.BlockSpec((1, 3 * D), lambda i: (0, 0)),
        ],
        out_specs=[
            pl.BlockSpec((BM, D), lambda i: (i, 0)),
            pl.BlockSpec((BM, D), lambda i: (i, 0)),
            pl.BlockSpec((BM, D), lambda i: (i, 0)),
            pl.BlockSpec((BM, D), lambda i: (i, 0)),
        ],
        out_shape=[
            jax.ShapeDtypeStruct((N, D), jnp.float32),
            jax.ShapeDtypeStruct((N, D), jnp.bfloat16),
            jax.ShapeDtypeStruct((N, D), jnp.bfloat16),
            jax.ShapeDtypeStruct((N, D), jnp.bfloat16),
        ],
        interpret=interpret,
    )(adj, x, x, gc_W, gc_b, root, in_w, in_b)


def _attn_gate_body(qb_ref, kb_ref, vb_ref, x1_ref, ow_ref, ob_ref, g1_ref,
                    b1_ref, gw_ref, x2p_ref, e0_ref, e1_ref,
                    w0_ref, w1_ref):
    bf = jnp.bfloat16
    kb = kb_ref[...]
    vb = vb_ref[...]
    cols = []
    for h in range(H):
        q = qb_ref[:, DH * h:DH * (h + 1)]
        k = kb[:, DH * h:DH * (h + 1)]
        v = vb[:, DH * h:DH * (h + 1)]
        sc = lax.dot_general(q, k, (((1,), (1,)), ((), ())),
                             preferred_element_type=jnp.float32)
        m = jnp.max(sc, axis=1, keepdims=True)
        p = jnp.exp(sc - m)
        inv = 1.0 / jnp.sum(p, axis=1, keepdims=True)
        cols.append(jnp.dot((p * inv).astype(bf), v,
                            preferred_element_type=jnp.float32))
    at = jnp.concatenate(cols, axis=1)
    attn = lax.dot_general(at.astype(bf), ow_ref[...].astype(bf),
                           (((1,), (1,)), ((), ())),
                           preferred_element_type=jnp.float32) + ob_ref[...]
    x2 = _ln(x1_ref[...] + attn, g1_ref[...], b1_ref[...])
    x2p_ref[...] = pltpu.bitcast(x2.astype(bf).reshape(2 * BM, D // 2), jnp.int32)
    logits = lax.dot_general(x2, gw_ref[...], (((1,), (1,)), ((), ())),
                             preferred_element_type=jnp.float32)
    lane = lax.broadcasted_iota(jnp.int32, logits.shape, 1)
    lg = jnp.where(lane < E, logits, NEG)
    m1 = jnp.max(lg, axis=1, keepdims=True)
    i1 = jnp.min(jnp.where(lg == m1, lane, 128), axis=1, keepdims=True)
    lg2 = jnp.where(lane == i1, NEG, lg)
    m2 = jnp.max(lg2, axis=1, keepdims=True)
    i2 = jnp.min(jnp.where(lg2 == m2, lane, 128), axis=1, keepdims=True)
    w1v = 1.0 / (1.0 + jnp.exp(m2 - m1))
    e0_ref[...] = i1
    e1_ref[...] = i2
    w0_ref[...] = w1v
    w1_ref[...] = 1.0 - w1v


def _attn_gate(qb, kb, vb, x1, out_w, out_b, ln1_g, ln1_b, gate_wp,
               interpret=False):
    return pl.pallas_call(
        _attn_gate_body,
        grid=(N // BM,),
        in_specs=[
            pl.BlockSpec((BM, D), lambda i: (i, 0)),
            pl.BlockSpec((N, D), lambda i: (0, 0)),
            pl.BlockSpec((N, D), lambda i: (0, 0)),
            pl.BlockSpec((BM, D), lambda i: (i, 0)),
            pl.BlockSpec((D, D), lambda i: (0, 0)),
            pl.BlockSpec((1, D), lambda i: (0, 0)),
            pl.BlockSpec((1, D), lambda i: (0, 0)),
            pl.BlockSpec((1, D), lambda i: (0, 0)),
            pl.BlockSpec((128, D), lambda i: (0, 0)),
        ],
        out_specs=[
            pl.BlockSpec((BM, D // 2), lambda i: (i, 0)),
            pl.BlockSpec((BM, 1), lambda i: (i, 0)),
            pl.BlockSpec((BM, 1), lambda i: (i, 0)),
            pl.BlockSpec((BM, 1), lambda i: (i, 0)),
            pl.BlockSpec((BM, 1), lambda i: (i, 0)),
        ],
        out_shape=[
            jax.ShapeDtypeStruct((N, D // 2), jnp.int32),
            jax.ShapeDtypeStruct((N, 1), jnp.int32),
            jax.ShapeDtypeStruct((N, 1), jnp.int32),
            jax.ShapeDtypeStruct((N, 1), jnp.float32),
            jax.ShapeDtypeStruct((N, 1), jnp.float32),
        ],
        interpret=interpret,
    )(qb, kb, vb, x1, out_w, out_b, ln1_g, ln1_b, gate_wp)


ROWS = SLOTS // 128


def _sortpos_body(eid_ref, pos_ref, texp_ref, used_ref, zidx_ref):
    eid = eid_ref[...]
    tri_r = lax.broadcasted_iota(jnp.int32, (ROWS, ROWS), 0)
    tri_c = lax.broadcasted_iota(jnp.int32, (ROWS, ROWS), 1)
    tri = (tri_r > tri_c).astype(jnp.float32)
    cs_r = lax.broadcasted_iota(jnp.int32, (128, 128), 0)
    cs_c = lax.broadcasted_iota(jnp.int32, (128, 128), 1)
    csm = (cs_r <= cs_c).astype(jnp.float32)
    pos = jnp.zeros(eid.shape, jnp.float32)
    off = jnp.zeros((1, 1), jnp.float32)
    offs = []
    pads = []
    for e in range(E):
        mask = (eid == e).astype(jnp.float32)
        rowcs = jnp.dot(mask, csm, preferred_element_type=jnp.float32)
        rowsum = rowcs[:, 127:128]
        row_off = jnp.dot(tri, rowsum, preferred_element_type=jnp.float32)
        rank = row_off + rowcs - mask
        pos = pos + mask * (off + rank)
        offs.append(off)
        cnt = jnp.sum(mask).reshape(1, 1)
        pad = jnp.ceil(cnt * (1.0 / GTILE)) * GTILE
        pads.append(pad)
        off = off + pad
    pos_ref[...] = pos.astype(jnp.int32)
    start = lax.broadcasted_iota(jnp.int32, (1, 128), 1).astype(jnp.float32) * GTILE
    te = jnp.zeros((1, 128), jnp.float32)
    for e in range(E):
        ind = jnp.logical_and(start >= offs[e], start < offs[e] + pads[e])
        te = te + ind.astype(jnp.float32) * e
    used = (start < off).astype(jnp.float32)
    te = te * used + (E - 1) * (1.0 - used)
    texp_ref[...] = te.astype(jnp.int32)
    used_ref[...] = used.astype(jnp.int32)
    lane = lax.broadcasted_iota(jnp.int32, (1, 128), 1).astype(jnp.float32)
    fu = off * (1.0 / GTILE)
    zidx_ref[...] = (lane * used + fu * (1.0 - used)).astype(jnp.int32)


def _sortpos(eid, interpret=False):
    return pl.pallas_call(
        _sortpos_body,
        grid=(1,),
        in_specs=[pl.BlockSpec((ROWS, 128), lambda i: (0, 0))],
        out_specs=[
            pl.BlockSpec((ROWS, 128), lambda i: (0, 0)),
            pl.BlockSpec((1, 128), lambda i: (0, 0)),
            pl.BlockSpec((1, 128), lambda i: (0, 0)),
            pl.BlockSpec((1, 128), lambda i: (0, 0)),
        ],
        out_shape=[
            jax.ShapeDtypeStruct((ROWS, 128), jnp.int32),
            jax.ShapeDtypeStruct((1, 128), jnp.int32),
            jax.ShapeDtypeStruct((1, 128), jnp.int32),
            jax.ShapeDtypeStruct((1, 128), jnp.int32),
        ],
        interpret=interpret,
    )(eid)


_NW = 32
_SPW = SLOTS // _NW


def _sc_dispatch_body(x2_hbm, tok_hbm, pos_hbm, z_hbm, tokv, posv, rows, sem):
    wid = lax.axis_index("s") * 2 + lax.axis_index("c")
    base = wid * _SPW
    pltpu.sync_copy(tok_hbm.at[pl.ds(base, _SPW)], tokv)
    pltpu.sync_copy(pos_hbm.at[pl.ds(base, _SPW)], posv)
    pltpu.async_copy(x2_hbm.at[tokv], rows, sem).wait()
    pltpu.async_copy(rows, z_hbm.at[posv], sem).wait()


def _sc_dispatch(x2, tok, pos):
    f = pl.kernel(
        _sc_dispatch_body,
        mesh=plsc.VectorSubcoreMesh(core_axis_name="c", subcore_axis_name="s"),
        out_type=jax.ShapeDtypeStruct((PAD, D // 2), jnp.int32),
        scratch_types=[
            pltpu.VMEM((_SPW,), jnp.int32),
            pltpu.VMEM((_SPW,), jnp.int32),
            pltpu.VMEM((_SPW, D // 2), jnp.int32),
            pltpu.SemaphoreType.DMA,
        ],
    )
    return f(x2, tok, pos)


def _sc_combine_body(y_hbm, pos_hbm, yp_hbm, posv, rows, sem):
    wid = lax.axis_index("s") * 2 + lax.axis_index("c")
    base = wid * _SPW
    pltpu.sync_copy(pos_hbm.at[pl.ds(base, _SPW)], posv)
    pltpu.async_copy(y_hbm.at[posv], rows, sem).wait()
    pltpu.sync_copy(rows, yp_hbm.at[pl.ds(base, _SPW)])


def _sc_combine(y, pos):
    f = pl.kernel(
        _sc_combine_body,
        mesh=plsc.VectorSubcoreMesh(core_axis_name="c", subcore_axis_name="s"),
        out_type=jax.ShapeDtypeStruct((SLOTS, D // 2), jnp.int32),
        scratch_types=[
            pltpu.VMEM((_SPW,), jnp.int32),
            pltpu.VMEM((_SPW, D // 2), jnp.int32),
            pltpu.SemaphoreType.DMA,
        ],
    )
    return f(y, pos)


def _expert_body(te_ref, used_ref, zi_ref, z_ref, w1_ref, b1_ref, w2_ref, b2_ref, y_ref):
    t = pl.program_id(0)

    @pl.when(used_ref[t] != 0)
    def _():
        bf = jnp.bfloat16
        z = pltpu.bitcast(z_ref[...], bf).reshape(GTILE, D)
        h = lax.dot_general(z, w1_ref[0].astype(bf), (((1,), (1,)), ((), ())),
                            preferred_element_type=jnp.float32) + b1_ref[0]
        h = 0.5 * h * (1.0 + lax.erf(h * (1.0 / math.sqrt(2.0))))
        y = lax.dot_general(h.astype(bf), w2_ref[0].astype(bf),
                            (((1,), (1,)), ((), ())),
                            preferred_element_type=jnp.float32) + b2_ref[0]
        y_ref[...] = pltpu.bitcast(y.astype(bf).reshape(2 * GTILE, D // 2),
                                   jnp.int32)


def _expert_ffn(z, texp, used, zidx, e_w1, e_b1, e_w2, e_b2, interpret=False):
    grid_spec = pltpu.PrefetchScalarGridSpec(
        num_scalar_prefetch=3,
        grid=(NTILES,),
        in_specs=[
            pl.BlockSpec((GTILE, D // 2), lambda t, te, u, zi: (zi[t], 0)),
            pl.BlockSpec((1, DFF, D), lambda t, te, u, zi: (te[t], 0, 0)),
            pl.BlockSpec((1, 1, DFF), lambda t, te, u, zi: (te[t], 0, 0)),
            pl.BlockSpec((1, D, DFF), lambda t, te, u, zi: (te[t], 0, 0)),
            pl.BlockSpec((1, 1, D), lambda t, te, u, zi: (te[t], 0, 0)),
        ],
        out_specs=pl.BlockSpec((GTILE, D // 2), lambda t, te, u, zi: (zi[t], 0)),
    )
    return pl.pallas_call(
        _expert_body,
        grid_spec=grid_spec,
        out_shape=jax.ShapeDtypeStruct((PAD, D // 2), jnp.int32),
        interpret=interpret,
    )(texp, used, zidx, z, e_w1, e_b1.reshape(E, 1, DFF), e_w2, e_b2.reshape(E, 1, D))


def _combine_body(x2p_ref, yp_ref, w0_ref, w1_ref, g2_ref, b2_ref, o_ref):
    unp = pltpu.bitcast(yp_ref[...], jnp.bfloat16)
    y0 = unp[:, :D // 2].reshape(BM, D).astype(jnp.float32)
    y1 = unp[:, D // 2:].reshape(BM, D).astype(jnp.float32)
    x2 = pltpu.bitcast(x2p_ref[...], jnp.bfloat16).reshape(BM, D).astype(jnp.float32)
    moe = w0_ref[...] * y0 + w1_ref[...] * y1
    o_ref[...] = _ln(x2 + moe, g2_ref[...], b2_ref[...])


def _combine(x2, ypairs, w0, w1, ln2_g, ln2_b, interpret=False):
    return pl.pallas_call(
        _combine_body,
        grid=(N // BM,),
        in_specs=[
            pl.BlockSpec((BM, D // 2), lambda i: (i, 0)),
            pl.BlockSpec((BM, D), lambda i: (i, 0)),
            pl.BlockSpec((BM, 1), lambda i: (i, 0)),
            pl.BlockSpec((BM, 1), lambda i: (i, 0)),
            pl.BlockSpec((1, D), lambda i: (0, 0)),
            pl.BlockSpec((1, D), lambda i: (0, 0)),
        ],
        out_specs=pl.BlockSpec((BM, D), lambda i: (i, 0)),
        out_shape=jax.ShapeDtypeStruct((N, D), jnp.float32),
        interpret=interpret,
    )(x2, ypairs, w0, w1, ln2_g, ln2_b)


def kernel(x, adj, gc_W, gc_b, root_emb, in_proj_w, in_proj_b, out_proj_w,
           out_proj_b, ln1_g, ln1_b, ln2_g, ln2_b, gate_w, e_w1, e_b1,
           e_w2, e_b2):
    xr = x[0]
    adjr = adj[0]
    x1, qb, kb, vb = _gconv_qkv(adjr, xr, gc_W, gc_b.reshape(1, D), root_emb,
                                in_proj_w, in_proj_b.reshape(1, 3 * D))
    gate_wp = jnp.zeros((128, D), jnp.float32).at[:E].set(gate_w)
    x2p, e0, e1, w0, w1 = _attn_gate(
        qb, kb, vb, x1, out_proj_w, out_proj_b.reshape(1, D),
        ln1_g.reshape(1, D), ln1_b.reshape(1, D), gate_wp)
    eid = jnp.concatenate([e0, e1], axis=1).reshape(ROWS, 128)
    pos, texp, used, zidx = _sortpos(eid)
    pos_flat = pos.reshape(SLOTS)
    tok = jnp.arange(SLOTS, dtype=jnp.int32) // TOPK
    z = _sc_dispatch(x2p, tok, pos_flat)
    y = _expert_ffn(z, texp.reshape(128), used.reshape(128), zidx.reshape(128),
                    e_w1, e_b1, e_w2, e_b2)
    ypairs = _sc_combine(y, pos_flat).reshape(N, D)
    out = _combine(x2p, ypairs, w0, w1, ln2_g.reshape(1, D), ln2_b.reshape(1, D))
    return out.reshape(1, N, D)


# speedup vs baseline: 1.0912x; 1.0912x over previous
import functools
import math

import jax
import jax.numpy as jnp
from jax import lax
from jax.experimental import pallas as pl
from jax.experimental.pallas import tpu as pltpu
from jax.experimental.pallas import tpu_sc as plsc

N = 2048
D = 768
E = 8
DFF = 1024
H = 12
DH = 64
TOPK = 2

BM = 512
GTILE = 256
SLOTS = N * TOPK
PAD = SLOTS + E * GTILE
NTILES = PAD // GTILE
NEG = -1e30


def _ln(x, g, b, eps=1e-5):
    m = jnp.mean(x, axis=-1, keepdims=True)
    v = jnp.mean((x - m) ** 2, axis=-1, keepdims=True)
    return (x - m) * lax.rsqrt(v + eps) * g + b


def _gconv_qkv_body(adj_ref, x_ref, xt_ref, gcw_ref, gcb_ref, root_ref,
                    inw_ref, inb_ref, x1_ref, qb_ref, kb_ref, vb_ref):
    bf = jnp.bfloat16
    a = jnp.dot(adj_ref[...].astype(bf), x_ref[...].astype(bf),
                preferred_element_type=jnp.float32)
    g = lax.dot_general(a.astype(bf), gcw_ref[...].astype(bf),
                        (((1,), (1,)), ((), ())),
                        preferred_element_type=jnp.float32)
    g = g + gcb_ref[...] + root_ref[...]
    x1 = xt_ref[...] + g
    x1_ref[...] = x1
    qkv = lax.dot_general(x1.astype(bf), inw_ref[...].astype(bf),
                          (((1,), (1,)), ((), ())),
                          preferred_element_type=jnp.float32) + inb_ref[...]
    qb_ref[...] = (qkv[:, :D] * (1.0 / math.sqrt(DH))).astype(bf)
    kb_ref[...] = qkv[:, D:2 * D].astype(bf)
    vb_ref[...] = qkv[:, 2 * D:].astype(bf)


def _gconv_qkv(adj, x, gc_W, gc_b, root, in_w, in_b, interpret=False):
    return pl.pallas_call(
        _gconv_qkv_body,
        grid=(N // BM,),
        in_specs=[
            pl.BlockSpec((BM, N), lambda i: (i, 0)),
            pl.BlockSpec((N, D), lambda i: (0, 0)),
            pl.BlockSpec((BM, D), lambda i: (i, 0)),
            pl.BlockSpec((D, D), lambda i: (0, 0)),
            pl.BlockSpec((1, D), lambda i: (0, 0)),
            pl.BlockSpec((1, D), lambda i: (0, 0)),
            pl.BlockSpec((3 * D, D), lambda i: (0, 0)),
            pl.BlockSpec((1, 3 * D), lambda i: (0, 0)),
        ],
        out_specs=[
            pl.BlockSpec((BM, D), lambda i: (i, 0)),
            pl.BlockSpec((BM, D), lambda i: (i, 0)),
            pl.BlockSpec((BM, D), lambda i: (i, 0)),
            pl.BlockSpec((BM, D), lambda i: (i, 0)),
        ],
        out_shape=[
            jax.ShapeDtypeStruct((N, D), jnp.float32),
            jax.ShapeDtypeStruct((N, D), jnp.bfloat16),
            jax.ShapeDtypeStruct((N, D), jnp.bfloat16),
            jax.ShapeDtypeStruct((N, D), jnp.bfloat16),
        ],
        interpret=interpret,
    )(adj, x, x, gc_W, gc_b, root, in_w, in_b)


def _attn_gate_body(qb_ref, kb_ref, vb_ref, x1_ref, ow_ref, ob_ref, g1_ref,
                    b1_ref, gw_ref, x2_ref, x2p_ref, e0_ref, e1_ref,
                    w0_ref, w1_ref):
    bf = jnp.bfloat16
    kb = kb_ref[...]
    vb = vb_ref[...]
    cols = []
    for h in range(H):
        q = qb_ref[:, DH * h:DH * (h + 1)]
        k = kb[:, DH * h:DH * (h + 1)]
        v = vb[:, DH * h:DH * (h + 1)]
        sc = lax.dot_general(q, k, (((1,), (1,)), ((), ())),
                             preferred_element_type=jnp.float32)
        m = jnp.max(sc, axis=1, keepdims=True)
        p = jnp.exp(sc - m)
        inv = 1.0 / jnp.sum(p, axis=1, keepdims=True)
        cols.append(jnp.dot(p.astype(bf), v,
                            preferred_element_type=jnp.float32) * inv)
    at = jnp.concatenate(cols, axis=1)
    attn = lax.dot_general(at.astype(bf), ow_ref[...].astype(bf),
                           (((1,), (1,)), ((), ())),
                           preferred_element_type=jnp.float32) + ob_ref[...]
    x2 = _ln(x1_ref[...] + attn, g1_ref[...], b1_ref[...])
    x2_ref[...] = x2
    x2p_ref[...] = pltpu.bitcast(x2.astype(bf).reshape(2 * BM, D // 2), jnp.int32)
    logits = lax.dot_general(x2, gw_ref[...], (((1,), (1,)), ((), ())),
                             preferred_element_type=jnp.float32)
    lane = lax.broadcasted_iota(jnp.int32, logits.shape, 1)
    lg = jnp.where(lane < E, logits, NEG)
    m1 = jnp.max(lg, axis=1, keepdims=True)
    i1 = jnp.min(jnp.where(lg == m1, lane, 128), axis=1, keepdims=True)
    lg2 = jnp.where(lane == i1, NEG, lg)
    m2 = jnp.max(lg2, axis=1, keepdims=True)
    i2 = jnp.min(jnp.where(lg2 == m2, lane, 128), axis=1, keepdims=True)
    w1v = 1.0 / (1.0 + jnp.exp(m2 - m1))
    e0_ref[...] = i1
    e1_ref[...] = i2
    w0_ref[...] = w1v
    w1_ref[...] = 1.0 - w1v


def _attn_gate(qb, kb, vb, x1, out_w, out_b, ln1_g, ln1_b, gate_wp,
               interpret=False):
    return pl.pallas_call(
        _attn_gate_body,
        grid=(N // BM,),
        in_specs=[
            pl.BlockSpec((BM, D), lambda i: (i, 0)),
            pl.BlockSpec((N, D), lambda i: (0, 0)),
            pl.BlockSpec((N, D), lambda i: (0, 0)),
            pl.BlockSpec((BM, D), lambda i: (i, 0)),
            pl.BlockSpec((D, D), lambda i: (0, 0)),
            pl.BlockSpec((1, D), lambda i: (0, 0)),
            pl.BlockSpec((1, D), lambda i: (0, 0)),
            pl.BlockSpec((1, D), lambda i: (0, 0)),
            pl.BlockSpec((128, D), lambda i: (0, 0)),
        ],
        out_specs=[
            pl.BlockSpec((BM, D), lambda i: (i, 0)),
            pl.BlockSpec((BM, D // 2), lambda i: (i, 0)),
            pl.BlockSpec((BM, 1), lambda i: (i, 0)),
            pl.BlockSpec((BM, 1), lambda i: (i, 0)),
            pl.BlockSpec((BM, 1), lambda i: (i, 0)),
            pl.BlockSpec((BM, 1), lambda i: (i, 0)),
        ],
        out_shape=[
            jax.ShapeDtypeStruct((N, D), jnp.float32),
            jax.ShapeDtypeStruct((N, D // 2), jnp.int32),
            jax.ShapeDtypeStruct((N, 1), jnp.int32),
            jax.ShapeDtypeStruct((N, 1), jnp.int32),
            jax.ShapeDtypeStruct((N, 1), jnp.float32),
            jax.ShapeDtypeStruct((N, 1), jnp.float32),
        ],
        interpret=interpret,
    )(qb, kb, vb, x1, out_w, out_b, ln1_g, ln1_b, gate_wp)


ROWS = SLOTS // 128


def _sortpos_body(eid_ref, pos_ref, texp_ref, used_ref, zidx_ref):
    eid = eid_ref[...]
    tri_r = lax.broadcasted_iota(jnp.int32, (ROWS, ROWS), 0)
    tri_c = lax.broadcasted_iota(jnp.int32, (ROWS, ROWS), 1)
    tri = (tri_r > tri_c).astype(jnp.float32)
    cs_r = lax.broadcasted_iota(jnp.int32, (128, 128), 0)
    cs_c = lax.broadcasted_iota(jnp.int32, (128, 128), 1)
    csm = (cs_r <= cs_c).astype(jnp.float32)
    pos = jnp.zeros(eid.shape, jnp.float32)
    off = jnp.zeros((1, 1), jnp.float32)
    offs = []
    pads = []
    for e in range(E):
        mask = (eid == e).astype(jnp.float32)
        rowcs = jnp.dot(mask, csm, preferred_element_type=jnp.float32)
        rowsum = rowcs[:, 127:128]
        row_off = jnp.dot(tri, rowsum, preferred_element_type=jnp.float32)
        rank = row_off + rowcs - mask
        pos = pos + mask * (off + rank)
        offs.append(off)
        cnt = jnp.sum(mask).reshape(1, 1)
        pad = jnp.ceil(cnt * (1.0 / GTILE)) * GTILE
        pads.append(pad)
        off = off + pad
    pos_ref[...] = pos.astype(jnp.int32)
    start = lax.broadcasted_iota(jnp.int32, (1, 128), 1).astype(jnp.float32) * GTILE
    te = jnp.zeros((1, 128), jnp.float32)
    for e in range(E):
        ind = jnp.logical_and(start >= offs[e], start < offs[e] + pads[e])
        te = te + ind.astype(jnp.float32) * e
    used = (start < off).astype(jnp.float32)
    te = te * used + (E - 1) * (1.0 - used)
    texp_ref[...] = te.astype(jnp.int32)
    used_ref[...] = used.astype(jnp.int32)
    lane = lax.broadcasted_iota(jnp.int32, (1, 128), 1).astype(jnp.float32)
    fu = off * (1.0 / GTILE)
    zidx_ref[...] = (lane * used + fu * (1.0 - used)).astype(jnp.int32)


def _sortpos(eid, interpret=False):
    return pl.pallas_call(
        _sortpos_body,
        grid=(1,),
        in_specs=[pl.BlockSpec((ROWS, 128), lambda i: (0, 0))],
        out_specs=[
            pl.BlockSpec((ROWS, 128), lambda i: (0, 0)),
            pl.BlockSpec((1, 128), lambda i: (0, 0)),
            pl.BlockSpec((1, 128), lambda i: (0, 0)),
            pl.BlockSpec((1, 128), lambda i: (0, 0)),
        ],
        out_shape=[
            jax.ShapeDtypeStruct((ROWS, 128), jnp.int32),
            jax.ShapeDtypeStruct((1, 128), jnp.int32),
            jax.ShapeDtypeStruct((1, 128), jnp.int32),
            jax.ShapeDtypeStruct((1, 128), jnp.int32),
        ],
        interpret=interpret,
    )(eid)


_NW = 32
_SPW = SLOTS // _NW


def _sc_dispatch_body(x2_hbm, tok_hbm, pos_hbm, z_hbm, tokv, posv, rows, sem):
    wid = lax.axis_index("s") * 2 + lax.axis_index("c")
    base = wid * _SPW
    pltpu.sync_copy(tok_hbm.at[pl.ds(base, _SPW)], tokv)
    pltpu.sync_copy(pos_hbm.at[pl.ds(base, _SPW)], posv)
    pltpu.async_copy(x2_hbm.at[tokv], rows, sem).wait()
    pltpu.async_copy(rows, z_hbm.at[posv], sem).wait()


def _sc_dispatch(x2, tok, pos):
    f = pl.kernel(
        _sc_dispatch_body,
        mesh=plsc.VectorSubcoreMesh(core_axis_name="c", subcore_axis_name="s"),
        out_type=jax.ShapeDtypeStruct((PAD, D // 2), jnp.int32),
        scratch_types=[
            pltpu.VMEM((_SPW,), jnp.int32),
            pltpu.VMEM((_SPW,), jnp.int32),
            pltpu.VMEM((_SPW, D // 2), jnp.int32),
            pltpu.SemaphoreType.DMA,
        ],
    )
    return f(x2, tok, pos)


def _sc_combine_body(y_hbm, pos_hbm, yp_hbm, posv, rows, sem):
    wid = lax.axis_index("s") * 2 + lax.axis_index("c")
    base = wid * _SPW
    pltpu.sync_copy(pos_hbm.at[pl.ds(base, _SPW)], posv)
    pltpu.async_copy(y_hbm.at[posv], rows, sem).wait()
    pltpu.sync_copy(rows, yp_hbm.at[pl.ds(base, _SPW)])


def _sc_combine(y, pos):
    f = pl.kernel(
        _sc_combine_body,
        mesh=plsc.VectorSubcoreMesh(core_axis_name="c", subcore_axis_name="s"),
        out_type=jax.ShapeDtypeStruct((SLOTS, D // 2), jnp.int32),
        scratch_types=[
            pltpu.VMEM((_SPW,), jnp.int32),
            pltpu.VMEM((_SPW, D // 2), jnp.int32),
            pltpu.SemaphoreType.DMA,
        ],
    )
    return f(y, pos)


def _expert_body(te_ref, used_ref, zi_ref, z_ref, w1_ref, b1_ref, w2_ref, b2_ref, y_ref):
    t = pl.program_id(0)

    @pl.when(used_ref[t] != 0)
    def _():
        bf = jnp.bfloat16
        z = pltpu.bitcast(z_ref[...], bf).reshape(GTILE, D)
        h = lax.dot_general(z, w1_ref[0].astype(bf), (((1,), (1,)), ((), ())),
                            preferred_element_type=jnp.float32) + b1_ref[0]
        h = 0.5 * h * (1.0 + lax.erf(h * (1.0 / math.sqrt(2.0))))
        y = lax.dot_general(h.astype(bf), w2_ref[0].astype(bf),
                            (((1,), (1,)), ((), ())),
                            preferred_element_type=jnp.float32) + b2_ref[0]
        y_ref[...] = pltpu.bitcast(y.astype(bf).reshape(2 * GTILE, D // 2),
                                   jnp.int32)


def _expert_ffn(z, texp, used, zidx, e_w1, e_b1, e_w2, e_b2, interpret=False):
    grid_spec = pltpu.PrefetchScalarGridSpec(
        num_scalar_prefetch=3,
        grid=(NTILES,),
        in_specs=[
            pl.BlockSpec((GTILE, D // 2), lambda t, te, u, zi: (zi[t], 0)),
            pl.BlockSpec((1, DFF, D), lambda t, te, u, zi: (te[t], 0, 0)),
            pl.BlockSpec((1, 1, DFF), lambda t, te, u, zi: (te[t], 0, 0)),
            pl.BlockSpec((1, D, DFF), lambda t, te, u, zi: (te[t], 0, 0)),
            pl.BlockSpec((1, 1, D), lambda t, te, u, zi: (te[t], 0, 0)),
        ],
        out_specs=pl.BlockSpec((GTILE, D // 2), lambda t, te, u, zi: (zi[t], 0)),
    )
    return pl.pallas_call(
        _expert_body,
        grid_spec=grid_spec,
        out_shape=jax.ShapeDtypeStruct((PAD, D // 2), jnp.int32),
        interpret=interpret,
    )(texp, used, zidx, z, e_w1, e_b1.reshape(E, 1, DFF), e_w2, e_b2.reshape(E, 1, D))


def _combine_body(x2_ref, yp_ref, w0_ref, w1_ref, g2_ref, b2_ref, o_ref):
    unp = pltpu.bitcast(yp_ref[...], jnp.bfloat16)
    y0 = unp[:, :D // 2].reshape(BM, D).astype(jnp.float32)
    y1 = unp[:, D // 2:].reshape(BM, D).astype(jnp.float32)
    moe = w0_ref[...] * y0 + w1_ref[...] * y1
    o_ref[...] = _ln(x2_ref[...] + moe, g2_ref[...], b2_ref[...])


def _combine(x2, ypairs, w0, w1, ln2_g, ln2_b, interpret=False):
    return pl.pallas_call(
        _combine_body,
        grid=(N // BM,),
        in_specs=[
            pl.BlockSpec((BM, D), lambda i: (i, 0)),
            pl.BlockSpec((BM, D), lambda i: (i, 0)),
            pl.BlockSpec((BM, 1), lambda i: (i, 0)),
            pl.BlockSpec((BM, 1), lambda i: (i, 0)),
            pl.BlockSpec((1, D), lambda i: (0, 0)),
            pl.BlockSpec((1, D), lambda i: (0, 0)),
        ],
        out_specs=pl.BlockSpec((BM, D), lambda i: (i, 0)),
        out_shape=jax.ShapeDtypeStruct((N, D), jnp.float32),
        interpret=interpret,
    )(x2, ypairs, w0, w1, ln2_g, ln2_b)


def kernel(x, adj, gc_W, gc_b, root_emb, in_proj_w, in_proj_b, out_proj_w,
           out_proj_b, ln1_g, ln1_b, ln2_g, ln2_b, gate_w, e_w1, e_b1,
           e_w2, e_b2):
    xr = x[0]
    adjr = adj[0]
    x1, qb, kb, vb = _gconv_qkv(adjr, xr, gc_W, gc_b.reshape(1, D), root_emb,
                                in_proj_w, in_proj_b.reshape(1, 3 * D))
    gate_wp = jnp.zeros((128, D), jnp.float32).at[:E].set(gate_w)
    x2, x2p, e0, e1, w0, w1 = _attn_gate(
        qb, kb, vb, x1, out_proj_w, out_proj_b.reshape(1, D),
        ln1_g.reshape(1, D), ln1_b.reshape(1, D), gate_wp)
    eid = jnp.concatenate([e0, e1], axis=1).reshape(ROWS, 128)
    pos, texp, used, zidx = _sortpos(eid)
    pos_flat = pos.reshape(SLOTS)
    tok = jnp.arange(SLOTS, dtype=jnp.int32) // TOPK
    z = _sc_dispatch(x2p, tok, pos_flat)
    y = _expert_ffn(z, texp.reshape(128), used.reshape(128), zidx.reshape(128),
                    e_w1, e_b1, e_w2, e_b2)
    ypairs = _sc_combine(y, pos_flat).reshape(N, D)
    out = _combine(x2, ypairs, w0, w1, ln2_g.reshape(1, D), ln2_b.reshape(1, D))
    return out.reshape(1, N, D)
